# baseline (speedup 1.0000x reference)
import functools

import jax
import jax.numpy as jnp
from jax import lax
from jax.experimental import pallas as pl
from jax.experimental.pallas import tpu as pltpu
from jax.experimental.pallas import tpu_sc as plsc

N = 10000
NP = 10240
E = 320000
EP = 327680
NC, NS = 2, 16
NW = NC * NS
EPT = EP // NW
SLAB = NP // NS

F1, F2, F3, F4 = 112, 64, 32, 16
IN_DIM = 128
OUT_DIM = 4
BR = 1024


def _make_sc_scatter(F, CH):
    mesh = plsc.VectorSubcoreMesh(core_axis_name="c", subcore_axis_name="s")
    NCHUNK = EPT // CH

    @functools.partial(
        pl.kernel,
        out_type=[jax.ShapeDtypeStruct((NP, F), jnp.float32),
                  jax.ShapeDtypeStruct((NP, F), jnp.float32)],
        mesh=mesh,
        compiler_params=pltpu.CompilerParams(use_tc_tiling_on_sc=False),
        scratch_types=[
            pltpu.VMEM((NCHUNK, CH), jnp.int32),
            pltpu.VMEM((NCHUNK, CH), jnp.int32),
            pltpu.VMEM((CH, F), jnp.float32),
            pltpu.VMEM((CH, F), jnp.float32),
            pltpu.VMEM_SHARED((NP, F), jnp.float32),
            pltpu.SemaphoreType.DMA,
            pltpu.SemaphoreType.DMA,
        ],
    )
    def k(src_hbm, dst_hbm, tab_hbm, zeros_hbm, out0_hbm, out1_hbm,
          src_v, dst_v, ra, rb, acc, sa, sb):
        c = lax.axis_index("c")
        s = lax.axis_index("s")
        wid = s * NC + c
        pltpu.sync_copy(src_hbm.at[wid], src_v)
        pltpu.sync_copy(dst_hbm.at[wid], dst_v)
        pltpu.sync_copy(zeros_hbm, acc.at[pl.ds(s * SLAB, SLAB)])
        plsc.subcore_barrier()
        pltpu.async_copy(tab_hbm.at[src_v.at[0]], ra, sa)

        def body(j2, carry):
            j = j2 * 2
            pltpu.async_copy(tab_hbm.at[src_v.at[j + 1]], rb, sb)
            pltpu.make_async_copy(tab_hbm.at[src_v.at[j]], ra, sa).wait()
            pltpu.sync_copy(ra, acc.at[dst_v.at[j]], add=True)

            @pl.when(j + 2 < NCHUNK)
            def _():
                pltpu.async_copy(tab_hbm.at[src_v.at[j + 2]], ra, sa)

            pltpu.make_async_copy(tab_hbm.at[src_v.at[j + 1]], rb, sb).wait()
            pltpu.sync_copy(rb, acc.at[dst_v.at[j + 1]], add=True)
            return carry

        lax.fori_loop(0, NCHUNK // 2, body, 0)
        plsc.subcore_barrier()

        @pl.when(c == 0)
        def _():
            pltpu.sync_copy(acc.at[pl.ds(s * SLAB, SLAB)],
                            out0_hbm.at[pl.ds(s * SLAB, SLAB)])

        @pl.when(c == 1)
        def _():
            pltpu.sync_copy(acc.at[pl.ds(s * SLAB, SLAB)],
                            out1_hbm.at[pl.ds(s * SLAB, SLAB)])

    return k


def _make_sc_degree(CH):
    mesh = plsc.VectorSubcoreMesh(core_axis_name="c", subcore_axis_name="s")
    NCHUNK = EPT // CH

    @functools.partial(
        pl.kernel,
        out_type=[jax.ShapeDtypeStruct((NP, 16), jnp.float32),
                  jax.ShapeDtypeStruct((NP, 16), jnp.float32)],
        mesh=mesh,
        compiler_params=pltpu.CompilerParams(use_tc_tiling_on_sc=False),
        scratch_types=[
            pltpu.VMEM((NCHUNK, CH), jnp.int32),
            pltpu.VMEM((CH, 16), jnp.float32),
            pltpu.VMEM_SHARED((NP, 16), jnp.float32),
        ],
    )
    def k(dst_hbm, ones_hbm, zeros_hbm, out0_hbm, out1_hbm, dst_v, ones_v, acc):
        c = lax.axis_index("c")
        s = lax.axis_index("s")
        wid = s * NC + c
        pltpu.sync_copy(dst_hbm.at[wid], dst_v)
        pltpu.sync_copy(ones_hbm, ones_v)
        pltpu.sync_copy(zeros_hbm, acc.at[pl.ds(s * SLAB, SLAB)])
        plsc.subcore_barrier()

        def body(j, carry):
            pltpu.sync_copy(ones_v, acc.at[dst_v.at[j]], add=True)
            return carry

        lax.fori_loop(0, NCHUNK, body, 0)
        plsc.subcore_barrier()

        @pl.when(c == 0)
        def _():
            pltpu.sync_copy(acc.at[pl.ds(s * SLAB, SLAB)],
                            out0_hbm.at[pl.ds(s * SLAB, SLAB)])

        @pl.when(c == 1)
        def _():
            pltpu.sync_copy(acc.at[pl.ds(s * SLAB, SLAB)],
                            out1_hbm.at[pl.ds(s * SLAB, SLAB)])

    return k


_SC_CH = {F1: 128, F2: 512, F3: 1024, F4: 1024}
_SC_SCATTER = {F: _make_sc_scatter(F, _SC_CH[F]) for F in (F1, F2, F3, F4)}
_DEG_CH = 512
_SC_DEGREE = _make_sc_degree(_DEG_CH)


def _tc_stage0(x, d0, d1, W):
    def body(x_ref, d0_ref, d1_ref, w_ref, hs_ref, dis_ref):
        deg = d0_ref[:, :1] + d1_ref[:, :1] + 1.0
        dis = lax.rsqrt(deg)
        dis_ref[...] = dis
        hs_ref[...] = jnp.dot(x_ref[...] * dis, w_ref[...],
                              preferred_element_type=jnp.float32)

    return pl.pallas_call(
        body,
        grid=(NP // BR,),
        in_specs=[
            pl.BlockSpec((BR, IN_DIM), lambda i: (i, 0)),
            pl.BlockSpec((BR, 16), lambda i: (i, 0)),
            pl.BlockSpec((BR, 16), lambda i: (i, 0)),
            pl.BlockSpec((IN_DIM, F1), lambda i: (0, 0)),
        ],
        out_specs=[
            pl.BlockSpec((BR, F1), lambda i: (i, 0)),
            pl.BlockSpec((BR, 1), lambda i: (i, 0)),
        ],
        out_shape=[
            jax.ShapeDtypeStruct((NP, F1), jnp.float32),
            jax.ShapeDtypeStruct((NP, 1), jnp.float32),
        ],
    )(x, d0, d1, W)


def _tc_stage_mid(a0, a1, hs, dis, b, W, Fi, Fo):
    def body(a0_ref, a1_ref, hs_ref, dis_ref, b_ref, w_ref, out_ref):
        dis = dis_ref[...]
        g = jnp.maximum(dis * (a0_ref[...] + a1_ref[...] + hs_ref[...])
                        + b_ref[...], 0.0)
        out_ref[...] = jnp.dot(g * dis, w_ref[...],
                               preferred_element_type=jnp.float32)

    return pl.pallas_call(
        body,
        grid=(NP // BR,),
        in_specs=[
            pl.BlockSpec((BR, Fi), lambda i: (i, 0)),
            pl.BlockSpec((BR, Fi), lambda i: (i, 0)),
            pl.BlockSpec((BR, Fi), lambda i: (i, 0)),
            pl.BlockSpec((BR, 1), lambda i: (i, 0)),
            pl.BlockSpec((1, Fi), lambda i: (0, 0)),
            pl.BlockSpec((Fi, Fo), lambda i: (0, 0)),
        ],
        out_specs=pl.BlockSpec((BR, Fo), lambda i: (i, 0)),
        out_shape=jax.ShapeDtypeStruct((NP, Fo), jnp.float32),
    )(a0, a1, hs, dis, b, W)


def _tc_stage_final(a0, a1, hs, dis, b):
    def body(a0_ref, a1_ref, hs_ref, dis_ref, b_ref, out_ref):
        out_ref[...] = (dis_ref[...]
                        * (a0_ref[...] + a1_ref[...] + hs_ref[...])
                        + b_ref[...])

    return pl.pallas_call(
        body,
        grid=(NP // BR,),
        in_specs=[
            pl.BlockSpec((BR, F4), lambda i: (i, 0)),
            pl.BlockSpec((BR, F4), lambda i: (i, 0)),
            pl.BlockSpec((BR, F4), lambda i: (i, 0)),
            pl.BlockSpec((BR, 1), lambda i: (i, 0)),
            pl.BlockSpec((1, F4), lambda i: (0, 0)),
        ],
        out_specs=pl.BlockSpec((BR, F4), lambda i: (i, 0)),
        out_shape=jax.ShapeDtypeStruct((NP, F4), jnp.float32),
    )(a0, a1, hs, dis, b)


def kernel(x, edge_index, W1, b1, W2, b2, W3, b3, W4, b4):
    ei = edge_index.astype(jnp.int32)
    pad = jnp.full((EP - E,), NP - 1, jnp.int32)
    src_flat = jnp.concatenate([ei[0], pad])
    dst_flat = jnp.concatenate([ei[1], pad])

    def _r(a, CH):
        return a.reshape(NW, EPT // CH, CH)

    W1p = jnp.pad(W1, ((0, 0), (0, F1 - W1.shape[1])))
    b1p = jnp.pad(b1, (0, F1 - b1.shape[0])).reshape(1, F1)
    W2p = jnp.pad(W2, ((0, F1 - W2.shape[0]), (0, 0)))
    b2p = b2.reshape(1, F2)
    W3p = W3
    b3p = b3.reshape(1, F3)
    W4p = jnp.pad(W4, ((0, 0), (0, F4 - W4.shape[1])))
    b4p = jnp.pad(b4, (0, F4 - b4.shape[0])).reshape(1, F4)

    ones16 = jnp.ones((_DEG_CH, 16), jnp.float32)
    zeros16 = jnp.zeros((SLAB, 16), jnp.float32)

    deg0, deg1 = _SC_DEGREE(_r(dst_flat, _DEG_CH), ones16, zeros16)
    xp = jnp.pad(x, ((0, NP - N), (0, 0)))
    hs1, dis = _tc_stage0(xp, deg0, deg1, W1p)

    def _scat(F, tab):
        return _SC_SCATTER[F](_r(src_flat, _SC_CH[F]), _r(dst_flat, _SC_CH[F]),
                              tab, jnp.zeros((SLAB, F), jnp.float32))

    a0, a1 = _scat(F1, hs1)
    hs2 = _tc_stage_mid(a0, a1, hs1, dis, b1p, W2p, F1, F2)

    a0, a1 = _scat(F2, hs2)
    hs3 = _tc_stage_mid(a0, a1, hs2, dis, b2p, W3p, F2, F3)

    a0, a1 = _scat(F3, hs3)
    hs4 = _tc_stage_mid(a0, a1, hs3, dis, b3p, W4p, F3, F4)

    a0, a1 = _scat(F4, hs4)
    out = _tc_stage_final(a0, a1, hs4, dis, b4p)
    return out[:N, :OUT_DIM]


# speedup vs baseline: 1.1368x; 1.1368x over previous
import functools

import jax
import jax.numpy as jnp
from jax import lax
from jax.experimental import pallas as pl
from jax.experimental.pallas import tpu as pltpu
from jax.experimental.pallas import tpu_sc as plsc

N = 10000
NP = 10240
E = 320000
EP = 327680
NC, NS = 2, 16
NW = NC * NS
EPT = EP // NW
SLAB = NP // NS

F1, F2, F3, F4 = 112, 64, 32, 16
IN_DIM = 128
OUT_DIM = 4
BR = 1024


def _make_sc_scatter(F, CH, stage_tab):
    mesh = plsc.VectorSubcoreMesh(core_axis_name="c", subcore_axis_name="s")
    NCHUNK = EPT // CH

    scratch = [
        pltpu.VMEM((NCHUNK, CH), jnp.int32),
        pltpu.VMEM((NCHUNK, CH), jnp.int32),
        pltpu.VMEM((CH, F), jnp.float32),
        pltpu.VMEM((CH, F), jnp.float32),
        pltpu.VMEM_SHARED((NP, F), jnp.float32),
        pltpu.SemaphoreType.DMA,
        pltpu.SemaphoreType.DMA,
    ]
    if stage_tab:
        scratch.append(pltpu.VMEM_SHARED((NP, F), jnp.float32))

    @functools.partial(
        pl.kernel,
        out_type=[jax.ShapeDtypeStruct((NP, F), jnp.float32),
                  jax.ShapeDtypeStruct((NP, F), jnp.float32)],
        mesh=mesh,
        compiler_params=pltpu.CompilerParams(use_tc_tiling_on_sc=False),
        scratch_types=scratch,
    )
    def k(src_hbm, dst_hbm, tab_hbm, zeros_hbm, out0_hbm, out1_hbm,
          src_v, dst_v, ra, rb, acc, sa, sb, *maybe_ltab):
        c = lax.axis_index("c")
        s = lax.axis_index("s")
        wid = s * NC + c
        pltpu.sync_copy(src_hbm.at[wid], src_v)
        pltpu.sync_copy(dst_hbm.at[wid], dst_v)
        pltpu.sync_copy(zeros_hbm, acc.at[pl.ds(s * SLAB, SLAB)])
        if stage_tab:
            tab = maybe_ltab[0]
            pltpu.sync_copy(tab_hbm.at[pl.ds(s * SLAB, SLAB)],
                            tab.at[pl.ds(s * SLAB, SLAB)])
        else:
            tab = tab_hbm
        plsc.subcore_barrier()
        pltpu.async_copy(tab.at[src_v.at[0]], ra, sa)

        def body(j2, carry):
            j = j2 * 2
            pltpu.async_copy(tab.at[src_v.at[j + 1]], rb, sb)
            pltpu.make_async_copy(tab.at[src_v.at[j]], ra, sa).wait()
            pltpu.sync_copy(ra, acc.at[dst_v.at[j]], add=True)

            @pl.when(j + 2 < NCHUNK)
            def _():
                pltpu.async_copy(tab.at[src_v.at[j + 2]], ra, sa)

            pltpu.make_async_copy(tab.at[src_v.at[j + 1]], rb, sb).wait()
            pltpu.sync_copy(rb, acc.at[dst_v.at[j + 1]], add=True)
            return carry

        lax.fori_loop(0, NCHUNK // 2, body, 0)
        plsc.subcore_barrier()

        @pl.when(c == 0)
        def _():
            pltpu.sync_copy(acc.at[pl.ds(s * SLAB, SLAB)],
                            out0_hbm.at[pl.ds(s * SLAB, SLAB)])

        @pl.when(c == 1)
        def _():
            pltpu.sync_copy(acc.at[pl.ds(s * SLAB, SLAB)],
                            out1_hbm.at[pl.ds(s * SLAB, SLAB)])

    return k


def _make_sc_degree(CH):
    mesh = plsc.VectorSubcoreMesh(core_axis_name="c", subcore_axis_name="s")
    NCHUNK = EPT // CH

    @functools.partial(
        pl.kernel,
        out_type=[jax.ShapeDtypeStruct((NP, 16), jnp.float32),
                  jax.ShapeDtypeStruct((NP, 16), jnp.float32)],
        mesh=mesh,
        compiler_params=pltpu.CompilerParams(use_tc_tiling_on_sc=False),
        scratch_types=[
            pltpu.VMEM((NCHUNK, CH), jnp.int32),
            pltpu.VMEM((CH, 16), jnp.float32),
            pltpu.VMEM_SHARED((NP, 16), jnp.float32),
        ],
    )
    def k(dst_hbm, ones_hbm, zeros_hbm, out0_hbm, out1_hbm, dst_v, ones_v, acc):
        c = lax.axis_index("c")
        s = lax.axis_index("s")
        wid = s * NC + c
        pltpu.sync_copy(dst_hbm.at[wid], dst_v)
        pltpu.sync_copy(ones_hbm, ones_v)
        pltpu.sync_copy(zeros_hbm, acc.at[pl.ds(s * SLAB, SLAB)])
        plsc.subcore_barrier()

        def body(j, carry):
            pltpu.sync_copy(ones_v, acc.at[dst_v.at[j]], add=True)
            return carry

        lax.fori_loop(0, NCHUNK, body, 0)
        plsc.subcore_barrier()

        @pl.when(c == 0)
        def _():
            pltpu.sync_copy(acc.at[pl.ds(s * SLAB, SLAB)],
                            out0_hbm.at[pl.ds(s * SLAB, SLAB)])

        @pl.when(c == 1)
        def _():
            pltpu.sync_copy(acc.at[pl.ds(s * SLAB, SLAB)],
                            out1_hbm.at[pl.ds(s * SLAB, SLAB)])

    return k


_SC_CH = {F1: 128, F2: 512, F3: 1024, F4: 1024}
_SC_STAGE = {F1: False, F2: False, F3: True, F4: True}
_SC_SCATTER = {F: _make_sc_scatter(F, _SC_CH[F], _SC_STAGE[F])
               for F in (F1, F2, F3, F4)}
_DEG_CH = 512
_SC_DEGREE = _make_sc_degree(_DEG_CH)


def _tc_stage0(x, d0, d1, W):
    def body(x_ref, d0_ref, d1_ref, w_ref, hs_ref, dis_ref):
        deg = d0_ref[:, :1] + d1_ref[:, :1] + 1.0
        dis = lax.rsqrt(deg)
        dis_ref[...] = dis
        hs_ref[...] = jnp.dot(x_ref[...] * dis, w_ref[...],
                              preferred_element_type=jnp.float32)

    return pl.pallas_call(
        body,
        grid=(NP // BR,),
        in_specs=[
            pl.BlockSpec((BR, IN_DIM), lambda i: (i, 0)),
            pl.BlockSpec((BR, 16), lambda i: (i, 0)),
            pl.BlockSpec((BR, 16), lambda i: (i, 0)),
            pl.BlockSpec((IN_DIM, F1), lambda i: (0, 0)),
        ],
        out_specs=[
            pl.BlockSpec((BR, F1), lambda i: (i, 0)),
            pl.BlockSpec((BR, 1), lambda i: (i, 0)),
        ],
        out_shape=[
            jax.ShapeDtypeStruct((NP, F1), jnp.float32),
            jax.ShapeDtypeStruct((NP, 1), jnp.float32),
        ],
    )(x, d0, d1, W)


def _tc_stage_mid(a0, a1, hs, dis, b, W, Fi, Fo):
    def body(a0_ref, a1_ref, hs_ref, dis_ref, b_ref, w_ref, out_ref):
        dis = dis_ref[...]
        g = jnp.maximum(dis * (a0_ref[...] + a1_ref[...] + hs_ref[...])
                        + b_ref[...], 0.0)
        out_ref[...] = jnp.dot(g * dis, w_ref[...],
                               preferred_element_type=jnp.float32)

    return pl.pallas_call(
        body,
        grid=(NP // BR,),
        in_specs=[
            pl.BlockSpec((BR, Fi), lambda i: (i, 0)),
            pl.BlockSpec((BR, Fi), lambda i: (i, 0)),
            pl.BlockSpec((BR, Fi), lambda i: (i, 0)),
            pl.BlockSpec((BR, 1), lambda i: (i, 0)),
            pl.BlockSpec((1, Fi), lambda i: (0, 0)),
            pl.BlockSpec((Fi, Fo), lambda i: (0, 0)),
        ],
        out_specs=pl.BlockSpec((BR, Fo), lambda i: (i, 0)),
        out_shape=jax.ShapeDtypeStruct((NP, Fo), jnp.float32),
    )(a0, a1, hs, dis, b, W)


def _tc_stage_final(a0, a1, hs, dis, b):
    def body(a0_ref, a1_ref, hs_ref, dis_ref, b_ref, out_ref):
        out_ref[...] = (dis_ref[...]
                        * (a0_ref[...] + a1_ref[...] + hs_ref[...])
                        + b_ref[...])

    return pl.pallas_call(
        body,
        grid=(NP // BR,),
        in_specs=[
            pl.BlockSpec((BR, F4), lambda i: (i, 0)),
            pl.BlockSpec((BR, F4), lambda i: (i, 0)),
            pl.BlockSpec((BR, F4), lambda i: (i, 0)),
            pl.BlockSpec((BR, 1), lambda i: (i, 0)),
            pl.BlockSpec((1, F4), lambda i: (0, 0)),
        ],
        out_specs=pl.BlockSpec((BR, F4), lambda i: (i, 0)),
        out_shape=jax.ShapeDtypeStruct((NP, F4), jnp.float32),
    )(a0, a1, hs, dis, b)


def kernel(x, edge_index, W1, b1, W2, b2, W3, b3, W4, b4):
    ei = edge_index.astype(jnp.int32)
    pad = jnp.full((EP - E,), NP - 1, jnp.int32)
    src_flat = jnp.concatenate([ei[0], pad])
    dst_flat = jnp.concatenate([ei[1], pad])

    def _r(a, CH):
        return a.reshape(NW, EPT // CH, CH)

    W1p = jnp.pad(W1, ((0, 0), (0, F1 - W1.shape[1])))
    b1p = jnp.pad(b1, (0, F1 - b1.shape[0])).reshape(1, F1)
    W2p = jnp.pad(W2, ((0, F1 - W2.shape[0]), (0, 0)))
    b2p = b2.reshape(1, F2)
    W3p = W3
    b3p = b3.reshape(1, F3)
    W4p = jnp.pad(W4, ((0, 0), (0, F4 - W4.shape[1])))
    b4p = jnp.pad(b4, (0, F4 - b4.shape[0])).reshape(1, F4)

    ones16 = jnp.ones((_DEG_CH, 16), jnp.float32)
    zeros16 = jnp.zeros((SLAB, 16), jnp.float32)

    deg0, deg1 = _SC_DEGREE(_r(dst_flat, _DEG_CH), ones16, zeros16)
    xp = jnp.pad(x, ((0, NP - N), (0, 0)))
    hs1, dis = _tc_stage0(xp, deg0, deg1, W1p)

    def _scat(F, tab):
        return _SC_SCATTER[F](_r(src_flat, _SC_CH[F]), _r(dst_flat, _SC_CH[F]),
                              tab, jnp.zeros((SLAB, F), jnp.float32))

    a0, a1 = _scat(F1, hs1)
    hs2 = _tc_stage_mid(a0, a1, hs1, dis, b1p, W2p, F1, F2)

    a0, a1 = _scat(F2, hs2)
    hs3 = _tc_stage_mid(a0, a1, hs2, dis, b2p, W3p, F2, F3)

    a0, a1 = _scat(F3, hs3)
    hs4 = _tc_stage_mid(a0, a1, hs3, dis, b3p, W4p, F3, F4)

    a0, a1 = _scat(F4, hs4)
    out = _tc_stage_final(a0, a1, hs4, dis, b4p)
    return out[:N, :OUT_DIM]


# speedup vs baseline: 1.4002x; 1.2317x over previous
import functools

import jax
import jax.numpy as jnp
from jax import lax
from jax.experimental import pallas as pl
from jax.experimental.pallas import tpu as pltpu
from jax.experimental.pallas import tpu_sc as plsc

N = 10000
NP = 10240
E = 320000
EP = 327680
NC, NS = 2, 16
NW = NC * NS
EPT = EP // NW
SLAB = NP // NS

F1, F2, F3, F4 = 112, 64, 32, 16
IN_DIM = 128
OUT_DIM = 4
BR = 1024


def _make_sc_scatter(F, CH, stage_tab):
    mesh = plsc.VectorSubcoreMesh(core_axis_name="c", subcore_axis_name="s")
    NCHUNK = EPT // CH

    scratch = [
        pltpu.VMEM((NCHUNK, CH), jnp.int32),
        pltpu.VMEM((NCHUNK, CH), jnp.int32),
        pltpu.VMEM((CH, F), jnp.float32),
        pltpu.VMEM((CH, F), jnp.float32),
        pltpu.VMEM_SHARED((NP, F), jnp.float32),
        pltpu.SemaphoreType.DMA,
        pltpu.SemaphoreType.DMA,
    ]
    if stage_tab:
        scratch.append(pltpu.VMEM_SHARED((NP, F), jnp.float32))

    @functools.partial(
        pl.kernel,
        out_type=[jax.ShapeDtypeStruct((NP, F), jnp.float32),
                  jax.ShapeDtypeStruct((NP, F), jnp.float32)],
        mesh=mesh,
        compiler_params=pltpu.CompilerParams(use_tc_tiling_on_sc=False),
        scratch_types=scratch,
    )
    def k(src_hbm, dst_hbm, tab_hbm, zeros_hbm, out0_hbm, out1_hbm,
          src_v, dst_v, ra, rb, acc, sa, sb, *maybe_ltab):
        c = lax.axis_index("c")
        s = lax.axis_index("s")
        wid = s * NC + c
        pltpu.sync_copy(src_hbm.at[wid], src_v)
        pltpu.sync_copy(dst_hbm.at[wid], dst_v)
        pltpu.sync_copy(zeros_hbm, acc.at[pl.ds(s * SLAB, SLAB)])
        if stage_tab:
            tab = maybe_ltab[0]
            pltpu.sync_copy(tab_hbm.at[pl.ds(s * SLAB, SLAB)],
                            tab.at[pl.ds(s * SLAB, SLAB)])
        else:
            tab = tab_hbm
        plsc.subcore_barrier()
        pltpu.async_copy(tab.at[src_v.at[0]], ra, sa)

        def body(j2, carry):
            j = j2 * 2
            pltpu.async_copy(tab.at[src_v.at[j + 1]], rb, sb)
            pltpu.make_async_copy(tab.at[src_v.at[j]], ra, sa).wait()
            pltpu.sync_copy(ra, acc.at[dst_v.at[j]], add=True)

            @pl.when(j + 2 < NCHUNK)
            def _():
                pltpu.async_copy(tab.at[src_v.at[j + 2]], ra, sa)

            pltpu.make_async_copy(tab.at[src_v.at[j + 1]], rb, sb).wait()
            pltpu.sync_copy(rb, acc.at[dst_v.at[j + 1]], add=True)
            return carry

        lax.fori_loop(0, NCHUNK // 2, body, 0)
        plsc.subcore_barrier()

        @pl.when(c == 0)
        def _():
            pltpu.sync_copy(acc.at[pl.ds(s * SLAB, SLAB)],
                            out0_hbm.at[pl.ds(s * SLAB, SLAB)])

        @pl.when(c == 1)
        def _():
            pltpu.sync_copy(acc.at[pl.ds(s * SLAB, SLAB)],
                            out1_hbm.at[pl.ds(s * SLAB, SLAB)])

    return k


def _make_sc_degree(CH):
    mesh = plsc.VectorSubcoreMesh(core_axis_name="c", subcore_axis_name="s")
    NCHUNK = EPT // CH

    @functools.partial(
        pl.kernel,
        out_type=[jax.ShapeDtypeStruct((NP, 16), jnp.float32),
                  jax.ShapeDtypeStruct((NP, 16), jnp.float32)],
        mesh=mesh,
        compiler_params=pltpu.CompilerParams(use_tc_tiling_on_sc=False),
        scratch_types=[
            pltpu.VMEM((NCHUNK, CH), jnp.int32),
            pltpu.VMEM((CH, 16), jnp.float32),
            pltpu.VMEM_SHARED((NP, 16), jnp.float32),
        ],
    )
    def k(dst_hbm, ones_hbm, zeros_hbm, out0_hbm, out1_hbm, dst_v, ones_v, acc):
        c = lax.axis_index("c")
        s = lax.axis_index("s")
        wid = s * NC + c
        pltpu.sync_copy(dst_hbm.at[wid], dst_v)
        pltpu.sync_copy(ones_hbm, ones_v)
        pltpu.sync_copy(zeros_hbm, acc.at[pl.ds(s * SLAB, SLAB)])
        plsc.subcore_barrier()

        def body(j, carry):
            pltpu.sync_copy(ones_v, acc.at[dst_v.at[j]], add=True)
            return carry

        lax.fori_loop(0, NCHUNK, body, 0)
        plsc.subcore_barrier()

        @pl.when(c == 0)
        def _():
            pltpu.sync_copy(acc.at[pl.ds(s * SLAB, SLAB)],
                            out0_hbm.at[pl.ds(s * SLAB, SLAB)])

        @pl.when(c == 1)
        def _():
            pltpu.sync_copy(acc.at[pl.ds(s * SLAB, SLAB)],
                            out1_hbm.at[pl.ds(s * SLAB, SLAB)])

    return k


_SC_CH = {F1: 128, F2: 128, F3: 1024, F4: 1024}
_SC_STAGE = {F1: False, F2: True, F3: True, F4: True}
_SC_SCATTER = {F: _make_sc_scatter(F, _SC_CH[F], _SC_STAGE[F])
               for F in (F1, F2, F3, F4)}
_DEG_CH = 512
_SC_DEGREE = _make_sc_degree(_DEG_CH)


def _tc_stage0(x, d0, d1, W):
    def body(x_ref, d0_ref, d1_ref, w_ref, hs_ref, dis_ref):
        deg = d0_ref[:, :1] + d1_ref[:, :1] + 1.0
        dis = lax.rsqrt(deg)
        dis_ref[...] = dis
        hs_ref[...] = jnp.dot(x_ref[...] * dis, w_ref[...],
                              preferred_element_type=jnp.float32)

    return pl.pallas_call(
        body,
        grid=(NP // BR,),
        in_specs=[
            pl.BlockSpec((BR, IN_DIM), lambda i: (i, 0)),
            pl.BlockSpec((BR, 16), lambda i: (i, 0)),
            pl.BlockSpec((BR, 16), lambda i: (i, 0)),
            pl.BlockSpec((IN_DIM, F1), lambda i: (0, 0)),
        ],
        out_specs=[
            pl.BlockSpec((BR, F1), lambda i: (i, 0)),
            pl.BlockSpec((BR, 1), lambda i: (i, 0)),
        ],
        out_shape=[
            jax.ShapeDtypeStruct((NP, F1), jnp.float32),
            jax.ShapeDtypeStruct((NP, 1), jnp.float32),
        ],
    )(x, d0, d1, W)


def _tc_stage_mid(a0, a1, hs, dis, b, W, Fi, Fo):
    def body(a0_ref, a1_ref, hs_ref, dis_ref, b_ref, w_ref, out_ref):
        dis = dis_ref[...]
        g = jnp.maximum(dis * (a0_ref[...] + a1_ref[...] + hs_ref[...])
                        + b_ref[...], 0.0)
        out_ref[...] = jnp.dot(g * dis, w_ref[...],
                               preferred_element_type=jnp.float32)

    return pl.pallas_call(
        body,
        grid=(NP // BR,),
        in_specs=[
            pl.BlockSpec((BR, Fi), lambda i: (i, 0)),
            pl.BlockSpec((BR, Fi), lambda i: (i, 0)),
            pl.BlockSpec((BR, Fi), lambda i: (i, 0)),
            pl.BlockSpec((BR, 1), lambda i: (i, 0)),
            pl.BlockSpec((1, Fi), lambda i: (0, 0)),
            pl.BlockSpec((Fi, Fo), lambda i: (0, 0)),
        ],
        out_specs=pl.BlockSpec((BR, Fo), lambda i: (i, 0)),
        out_shape=jax.ShapeDtypeStruct((NP, Fo), jnp.float32),
    )(a0, a1, hs, dis, b, W)


def _tc_stage_final(a0, a1, hs, dis, b):
    def body(a0_ref, a1_ref, hs_ref, dis_ref, b_ref, out_ref):
        out_ref[...] = (dis_ref[...]
                        * (a0_ref[...] + a1_ref[...] + hs_ref[...])
                        + b_ref[...])

    return pl.pallas_call(
        body,
        grid=(NP // BR,),
        in_specs=[
            pl.BlockSpec((BR, F4), lambda i: (i, 0)),
            pl.BlockSpec((BR, F4), lambda i: (i, 0)),
            pl.BlockSpec((BR, F4), lambda i: (i, 0)),
            pl.BlockSpec((BR, 1), lambda i: (i, 0)),
            pl.BlockSpec((1, F4), lambda i: (0, 0)),
        ],
        out_specs=pl.BlockSpec((BR, F4), lambda i: (i, 0)),
        out_shape=jax.ShapeDtypeStruct((NP, F4), jnp.float32),
    )(a0, a1, hs, dis, b)


def kernel(x, edge_index, W1, b1, W2, b2, W3, b3, W4, b4):
    ei = edge_index.astype(jnp.int32)
    pad = jnp.full((EP - E,), NP - 1, jnp.int32)
    src_flat = jnp.concatenate([ei[0], pad])
    dst_flat = jnp.concatenate([ei[1], pad])

    def _r(a, CH):
        return a.reshape(NW, EPT // CH, CH)

    W1p = jnp.pad(W1, ((0, 0), (0, F1 - W1.shape[1])))
    b1p = jnp.pad(b1, (0, F1 - b1.shape[0])).reshape(1, F1)
    W2p = jnp.pad(W2, ((0, F1 - W2.shape[0]), (0, 0)))
    b2p = b2.reshape(1, F2)
    W3p = W3
    b3p = b3.reshape(1, F3)
    W4p = jnp.pad(W4, ((0, 0), (0, F4 - W4.shape[1])))
    b4p = jnp.pad(b4, (0, F4 - b4.shape[0])).reshape(1, F4)

    ones16 = jnp.ones((_DEG_CH, 16), jnp.float32)
    zeros16 = jnp.zeros((SLAB, 16), jnp.float32)

    deg0, deg1 = _SC_DEGREE(_r(dst_flat, _DEG_CH), ones16, zeros16)
    xp = jnp.pad(x, ((0, NP - N), (0, 0)))
    hs1, dis = _tc_stage0(xp, deg0, deg1, W1p)

    def _scat(F, tab):
        return _SC_SCATTER[F](_r(src_flat, _SC_CH[F]), _r(dst_flat, _SC_CH[F]),
                              tab, jnp.zeros((SLAB, F), jnp.float32))

    a0, a1 = _scat(F1, hs1)
    hs2 = _tc_stage_mid(a0, a1, hs1, dis, b1p, W2p, F1, F2)

    a0, a1 = _scat(F2, hs2)
    hs3 = _tc_stage_mid(a0, a1, hs2, dis, b2p, W3p, F2, F3)

    a0, a1 = _scat(F3, hs3)
    hs4 = _tc_stage_mid(a0, a1, hs3, dis, b3p, W4p, F3, F4)

    a0, a1 = _scat(F4, hs4)
    out = _tc_stage_final(a0, a1, hs4, dis, b4p)
    return out[:N, :OUT_DIM]


# speedup vs baseline: 2.1461x; 1.5327x over previous
import functools

import jax
import jax.numpy as jnp
from jax import lax
from jax.experimental import pallas as pl
from jax.experimental.pallas import tpu as pltpu
from jax.experimental.pallas import tpu_sc as plsc

N = 10000
NP = 10240
E = 320000
EP = 327680
NC, NS = 2, 16
NW = NC * NS
EPT = EP // NW
SLAB = NP // NS

F1, F2, F3, F4 = 112, 64, 32, 16
IN_DIM = 128
OUT_DIM = 4
BR = 1024


def _make_sc_scatter(F, CH, stage_tab):
    mesh = plsc.VectorSubcoreMesh(core_axis_name="c", subcore_axis_name="s")
    NCHUNK = EPT // CH

    scratch = [
        pltpu.VMEM((NCHUNK, CH), jnp.int32),
        pltpu.VMEM((NCHUNK, CH), jnp.int32),
        pltpu.VMEM((CH, F), jnp.float32),
        pltpu.VMEM((CH, F), jnp.float32),
        pltpu.VMEM_SHARED((NP, F), jnp.float32),
        pltpu.SemaphoreType.DMA,
        pltpu.SemaphoreType.DMA,
    ]
    if stage_tab:
        scratch.append(pltpu.VMEM_SHARED((NP, F), jnp.float32))

    @functools.partial(
        pl.kernel,
        out_type=[jax.ShapeDtypeStruct((NP, F), jnp.float32),
                  jax.ShapeDtypeStruct((NP, F), jnp.float32)],
        mesh=mesh,
        compiler_params=pltpu.CompilerParams(use_tc_tiling_on_sc=False),
        scratch_types=scratch,
    )
    def k(src_hbm, dst_hbm, tab_hbm, zeros_hbm, out0_hbm, out1_hbm,
          src_v, dst_v, ra, rb, acc, sa, sb, *maybe_ltab):
        c = lax.axis_index("c")
        s = lax.axis_index("s")
        wid = s * NC + c
        pltpu.sync_copy(src_hbm.at[wid], src_v)
        pltpu.sync_copy(dst_hbm.at[wid], dst_v)
        pltpu.sync_copy(zeros_hbm, acc.at[pl.ds(s * SLAB, SLAB)])
        if stage_tab:
            tab = maybe_ltab[0]
            pltpu.sync_copy(tab_hbm.at[pl.ds(s * SLAB, SLAB)],
                            tab.at[pl.ds(s * SLAB, SLAB)])
        else:
            tab = tab_hbm
        plsc.subcore_barrier()
        pltpu.async_copy(tab.at[src_v.at[0]], ra, sa)

        def body(j2, carry):
            j = j2 * 2
            pltpu.async_copy(tab.at[src_v.at[j + 1]], rb, sb)
            pltpu.make_async_copy(tab.at[src_v.at[j]], ra, sa).wait()
            pltpu.sync_copy(ra, acc.at[dst_v.at[j]], add=True)

            @pl.when(j + 2 < NCHUNK)
            def _():
                pltpu.async_copy(tab.at[src_v.at[j + 2]], ra, sa)

            pltpu.make_async_copy(tab.at[src_v.at[j + 1]], rb, sb).wait()
            pltpu.sync_copy(rb, acc.at[dst_v.at[j + 1]], add=True)
            return carry

        lax.fori_loop(0, NCHUNK // 2, body, 0)
        plsc.subcore_barrier()

        @pl.when(c == 0)
        def _():
            pltpu.sync_copy(acc.at[pl.ds(s * SLAB, SLAB)],
                            out0_hbm.at[pl.ds(s * SLAB, SLAB)])

        @pl.when(c == 1)
        def _():
            pltpu.sync_copy(acc.at[pl.ds(s * SLAB, SLAB)],
                            out1_hbm.at[pl.ds(s * SLAB, SLAB)])

    return k


def _make_sc_degree(CH):
    mesh = plsc.VectorSubcoreMesh(core_axis_name="c", subcore_axis_name="s")
    NCHUNK = EPT // CH

    @functools.partial(
        pl.kernel,
        out_type=[jax.ShapeDtypeStruct((NP, 16), jnp.float32),
                  jax.ShapeDtypeStruct((NP, 16), jnp.float32)],
        mesh=mesh,
        compiler_params=pltpu.CompilerParams(use_tc_tiling_on_sc=False),
        scratch_types=[
            pltpu.VMEM((NCHUNK, CH), jnp.int32),
            pltpu.VMEM((CH, 16), jnp.float32),
            pltpu.VMEM_SHARED((NP, 16), jnp.float32),
        ],
    )
    def k(dst_hbm, ones_hbm, zeros_hbm, out0_hbm, out1_hbm, dst_v, ones_v, acc):
        c = lax.axis_index("c")
        s = lax.axis_index("s")
        wid = s * NC + c
        pltpu.sync_copy(dst_hbm.at[wid], dst_v)
        pltpu.sync_copy(ones_hbm, ones_v)
        pltpu.sync_copy(zeros_hbm, acc.at[pl.ds(s * SLAB, SLAB)])
        plsc.subcore_barrier()

        def body(j, carry):
            pltpu.sync_copy(ones_v, acc.at[dst_v.at[j]], add=True)
            return carry

        lax.fori_loop(0, NCHUNK, body, 0)
        plsc.subcore_barrier()

        @pl.when(c == 0)
        def _():
            pltpu.sync_copy(acc.at[pl.ds(s * SLAB, SLAB)],
                            out0_hbm.at[pl.ds(s * SLAB, SLAB)])

        @pl.when(c == 1)
        def _():
            pltpu.sync_copy(acc.at[pl.ds(s * SLAB, SLAB)],
                            out1_hbm.at[pl.ds(s * SLAB, SLAB)])

    return k


FH = F1 // 2
_SC_CH = {FH: 128, F2: 128, F3: 1024, F4: 1024}
_SC_STAGE = {FH: True, F2: True, F3: True, F4: True}
_SC_SCATTER = {F: _make_sc_scatter(F, _SC_CH[F], _SC_STAGE[F])
               for F in (FH, F2, F3, F4)}
_DEG_CH = 512
_SC_DEGREE = _make_sc_degree(_DEG_CH)


def _tc_stage0(x, d0, d1, W):
    def body(x_ref, d0_ref, d1_ref, w_ref, hsa_ref, hsb_ref, dis_ref):
        deg = d0_ref[:, :1] + d1_ref[:, :1] + 1.0
        dis = lax.rsqrt(deg)
        dis_ref[...] = dis
        hs = jnp.dot(x_ref[...] * dis, w_ref[...],
                     preferred_element_type=jnp.float32)
        hsa_ref[...] = hs[:, :FH]
        hsb_ref[...] = hs[:, FH:]

    return pl.pallas_call(
        body,
        grid=(NP // BR,),
        in_specs=[
            pl.BlockSpec((BR, IN_DIM), lambda i: (i, 0)),
            pl.BlockSpec((BR, 16), lambda i: (i, 0)),
            pl.BlockSpec((BR, 16), lambda i: (i, 0)),
            pl.BlockSpec((IN_DIM, F1), lambda i: (0, 0)),
        ],
        out_specs=[
            pl.BlockSpec((BR, FH), lambda i: (i, 0)),
            pl.BlockSpec((BR, FH), lambda i: (i, 0)),
            pl.BlockSpec((BR, 1), lambda i: (i, 0)),
        ],
        out_shape=[
            jax.ShapeDtypeStruct((NP, FH), jnp.float32),
            jax.ShapeDtypeStruct((NP, FH), jnp.float32),
            jax.ShapeDtypeStruct((NP, 1), jnp.float32),
        ],
    )(x, d0, d1, W)


def _tc_stage_mid1(a0a, a1a, a0b, a1b, hsa, hsb, dis, b, W):
    def body(a0a_ref, a1a_ref, a0b_ref, a1b_ref, hsa_ref, hsb_ref,
             dis_ref, b_ref, w_ref, out_ref):
        dis = dis_ref[...]
        ga = a0a_ref[...] + a1a_ref[...] + hsa_ref[...]
        gb = a0b_ref[...] + a1b_ref[...] + hsb_ref[...]
        g = jnp.maximum(dis * jnp.concatenate([ga, gb], axis=1)
                        + b_ref[...], 0.0)
        out_ref[...] = jnp.dot(g * dis, w_ref[...],
                               preferred_element_type=jnp.float32)

    half = pl.BlockSpec((BR, FH), lambda i: (i, 0))
    return pl.pallas_call(
        body,
        grid=(NP // BR,),
        in_specs=[
            half, half, half, half, half, half,
            pl.BlockSpec((BR, 1), lambda i: (i, 0)),
            pl.BlockSpec((1, F1), lambda i: (0, 0)),
            pl.BlockSpec((F1, F2), lambda i: (0, 0)),
        ],
        out_specs=pl.BlockSpec((BR, F2), lambda i: (i, 0)),
        out_shape=jax.ShapeDtypeStruct((NP, F2), jnp.float32),
    )(a0a, a1a, a0b, a1b, hsa, hsb, dis, b, W)


def _tc_stage_mid(a0, a1, hs, dis, b, W, Fi, Fo):
    def body(a0_ref, a1_ref, hs_ref, dis_ref, b_ref, w_ref, out_ref):
        dis = dis_ref[...]
        g = jnp.maximum(dis * (a0_ref[...] + a1_ref[...] + hs_ref[...])
                        + b_ref[...], 0.0)
        out_ref[...] = jnp.dot(g * dis, w_ref[...],
                               preferred_element_type=jnp.float32)

    return pl.pallas_call(
        body,
        grid=(NP // BR,),
        in_specs=[
            pl.BlockSpec((BR, Fi), lambda i: (i, 0)),
            pl.BlockSpec((BR, Fi), lambda i: (i, 0)),
            pl.BlockSpec((BR, Fi), lambda i: (i, 0)),
            pl.BlockSpec((BR, 1), lambda i: (i, 0)),
            pl.BlockSpec((1, Fi), lambda i: (0, 0)),
            pl.BlockSpec((Fi, Fo), lambda i: (0, 0)),
        ],
        out_specs=pl.BlockSpec((BR, Fo), lambda i: (i, 0)),
        out_shape=jax.ShapeDtypeStruct((NP, Fo), jnp.float32),
    )(a0, a1, hs, dis, b, W)


def _tc_stage_final(a0, a1, hs, dis, b):
    def body(a0_ref, a1_ref, hs_ref, dis_ref, b_ref, out_ref):
        out_ref[...] = (dis_ref[...]
                        * (a0_ref[...] + a1_ref[...] + hs_ref[...])
                        + b_ref[...])

    return pl.pallas_call(
        body,
        grid=(NP // BR,),
        in_specs=[
            pl.BlockSpec((BR, F4), lambda i: (i, 0)),
            pl.BlockSpec((BR, F4), lambda i: (i, 0)),
            pl.BlockSpec((BR, F4), lambda i: (i, 0)),
            pl.BlockSpec((BR, 1), lambda i: (i, 0)),
            pl.BlockSpec((1, F4), lambda i: (0, 0)),
        ],
        out_specs=pl.BlockSpec((BR, F4), lambda i: (i, 0)),
        out_shape=jax.ShapeDtypeStruct((NP, F4), jnp.float32),
    )(a0, a1, hs, dis, b)


def kernel(x, edge_index, W1, b1, W2, b2, W3, b3, W4, b4):
    ei = edge_index.astype(jnp.int32)
    pad = jnp.full((EP - E,), NP - 1, jnp.int32)
    src_flat = jnp.concatenate([ei[0], pad])
    dst_flat = jnp.concatenate([ei[1], pad])

    def _r(a, CH):
        return a.reshape(NW, EPT // CH, CH)

    W1p = jnp.pad(W1, ((0, 0), (0, F1 - W1.shape[1])))
    b1p = jnp.pad(b1, (0, F1 - b1.shape[0])).reshape(1, F1)
    W2p = jnp.pad(W2, ((0, F1 - W2.shape[0]), (0, 0)))
    b2p = b2.reshape(1, F2)
    W3p = W3
    b3p = b3.reshape(1, F3)
    W4p = jnp.pad(W4, ((0, 0), (0, F4 - W4.shape[1])))
    b4p = jnp.pad(b4, (0, F4 - b4.shape[0])).reshape(1, F4)

    ones16 = jnp.ones((_DEG_CH, 16), jnp.float32)
    zeros16 = jnp.zeros((SLAB, 16), jnp.float32)

    deg0, deg1 = _SC_DEGREE(_r(dst_flat, _DEG_CH), ones16, zeros16)
    xp = jnp.pad(x, ((0, NP - N), (0, 0)))
    hs1a, hs1b, dis = _tc_stage0(xp, deg0, deg1, W1p)

    def _scat(F, tab):
        return _SC_SCATTER[F](_r(src_flat, _SC_CH[F]), _r(dst_flat, _SC_CH[F]),
                              tab, jnp.zeros((SLAB, F), jnp.float32))

    a0a, a1a = _scat(FH, hs1a)
    a0b, a1b = _scat(FH, hs1b)
    hs2 = _tc_stage_mid1(a0a, a1a, a0b, a1b, hs1a, hs1b, dis, b1p, W2p)

    a0, a1 = _scat(F2, hs2)
    hs3 = _tc_stage_mid(a0, a1, hs2, dis, b2p, W3p, F2, F3)

    a0, a1 = _scat(F3, hs3)
    hs4 = _tc_stage_mid(a0, a1, hs3, dis, b3p, W4p, F3, F4)

    a0, a1 = _scat(F4, hs4)
    out = _tc_stage_final(a0, a1, hs4, dis, b4p)
    return out[:N, :OUT_DIM]
